# baseline (device time: 44048 ns/iter reference)
import jax
import jax.numpy as jnp
from jax import lax
from jax.experimental import pallas as pl
from jax.experimental.pallas import tpu as pltpu

N_DEV = 8
M = 1536
N = 1536
P_ROWS = M // 3
C = 3
CN = N // C
BUF_ROWS = (256, 128, 128)
SEMS_PER_CHUNK = 15


def kernel(A, B):
    def body(a_ref, b_ref, out_ref, zacc, abf, bbf, *rest):
        nbuf = C * 9
        flat_s, flat_r = rest[:nbuf], rest[nbuf:2 * nbuf]
        send_sems, recv_sems = rest[2 * nbuf], rest[2 * nbuf + 1]
        sbufs = [[[flat_s[(c * 3 + p) * 3 + k] for k in range(3)]
                  for p in range(3)] for c in range(C)]
        rbufs = [[[flat_r[(c * 3 + p) * 3 + k] for k in range(3)]
                  for p in range(3)] for c in range(C)]

        my = lax.axis_index("i")
        cx = (my ^ (my >> 1)) & 1
        cy = (my >> 1) & 1
        cz = (my >> 2) & 1
        ax_x = (my ^ 1, cx)
        ax_y = (my ^ 3, cy)
        ax_z = (my ^ 4, cz)
        orders = ((ax_x, ax_y, ax_z), (ax_y, ax_z, ax_x), (ax_z, ax_x, ax_y))

        bf16 = jnp.bfloat16
        f32 = jnp.float32

        def cslice(c):
            return pl.ds(c * CN, CN)

        def rs_rdma(c, p, k):
            partner, _ = orders[p][k]
            return pltpu.make_async_remote_copy(
                src_ref=sbufs[c][p][k],
                dst_ref=rbufs[c][p][k],
                send_sem=send_sems.at[c * SEMS_PER_CHUNK + 3 * k + p],
                recv_sem=recv_sems.at[c * SEMS_PER_CHUNK + 3 * k + p],
                device_id=(partner,),
                device_id_type=pl.DeviceIdType.MESH,
            )

        def ag_rdma(c, p, k, off, rows):
            partner, _ = orders[p][1 - k]
            return pltpu.make_async_remote_copy(
                src_ref=out_ref.at[pl.ds(off, rows), cslice(c)],
                dst_ref=out_ref.at[pl.ds(off, rows), cslice(c)],
                send_sem=send_sems.at[c * SEMS_PER_CHUNK + 9 + 3 * k + p],
                recv_sem=recv_sems.at[c * SEMS_PER_CHUNK + 9 + 3 * k + p],
                device_id=(partner,),
                device_id_type=pl.DeviceIdType.MESH,
            )

        barrier_sem = pltpu.get_barrier_semaphore()
        for nbr, _ in (ax_x, ax_y, ax_z):
            pl.semaphore_signal(
                barrier_sem, inc=1,
                device_id=(nbr,), device_id_type=pl.DeviceIdType.MESH,
            )
        pl.semaphore_wait(barrier_sem, 3)

        abf[...] = a_ref[...].astype(bf16)
        bbf[...] = b_ref[...].astype(bf16)

        rdmas = [[None] * 3 for _ in range(C)]
        seg = [[0] * 3 for _ in range(C)]

        def send_dots(c):
            for p in range(3):
                _, bit = orders[p][0]
                send_off = P_ROWS * p + (1 - bit) * 256
                seg[c][p] = P_ROWS * p + bit * 256
                sbufs[c][p][0][...] = jnp.dot(
                    abf[pl.ds(send_off, 256), :],
                    bbf[:, c * CN:(c + 1) * CN],
                    preferred_element_type=f32,
                ).astype(bf16)
                rdmas[c][p] = rs_rdma(c, p, 0)
                rdmas[c][p].start()

        def keep_dots(c):
            for p in range(3):
                zacc[pl.ds(seg[c][p], 256), cslice(c)] = jnp.dot(
                    abf[pl.ds(seg[c][p], 256), :],
                    bbf[:, c * CN:(c + 1) * CN],
                    preferred_element_type=f32,
                )

        def rs_step(c):
            for p in range(3):
                rdmas[c][p].wait()
                _, bit = orders[p][1]
                sbufs[c][p][1][...] = (
                    zacc[pl.ds(seg[c][p] + (1 - bit) * 128, 128), cslice(c)]
                    + rbufs[c][p][0][pl.ds((1 - bit) * 128, 128), :]
                    .astype(f32)
                ).astype(bf16)
                rdmas[c][p] = rs_rdma(c, p, 1)
                rdmas[c][p].start()
            for p in range(3):
                _, bit = orders[p][1]
                keep_off = seg[c][p] + bit * 128
                zacc[pl.ds(keep_off, 128), cslice(c)] = (
                    zacc[pl.ds(keep_off, 128), cslice(c)]
                    + rbufs[c][p][0][pl.ds(bit * 128, 128), :].astype(f32)
                )
                seg[c][p] = keep_off

        def merge_send(c):
            for p in range(3):
                rdmas[c][p].wait()
                sbufs[c][p][2][...] = (
                    zacc[pl.ds(seg[c][p], 128), cslice(c)]
                    + rbufs[c][p][1][...].astype(f32)
                ).astype(bf16)
                rdmas[c][p] = rs_rdma(c, p, 2)
                rdmas[c][p].start()

        def merge_silu(c):
            for p in range(3):
                rdmas[c][p].wait()
                z = (
                    sbufs[c][p][2][...].astype(f32)
                    + rbufs[c][p][2][...].astype(f32)
                )
                out_ref[pl.ds(seg[c][p], 128), cslice(c)] = (
                    z / (1.0 + jnp.exp(-z))
                ).astype(bf16)
                rdmas[c][p] = ag_rdma(c, p, 0, seg[c][p], 128)
                rdmas[c][p].start()

        def ag_step(c):
            for p in range(3):
                rdmas[c][p].wait()
                _, bit = orders[p][1]
                seg[c][p] = seg[c][p] - bit * 128
                rdmas[c][p] = ag_rdma(c, p, 1, seg[c][p], 256)
                rdmas[c][p].start()

        def ag_final(c):
            for p in range(3):
                rdmas[c][p].wait()

        steps = (
            send_dots,
            keep_dots,
            rs_step,
            merge_send,
            merge_silu,
            ag_step,
            ag_final,
        )
        for step in steps:
            for c in range(C):
                step(c)

    scratch = [
        pltpu.VMEM((M, N), jnp.float32),
        pltpu.VMEM((M, 768), jnp.bfloat16),
        pltpu.VMEM((768, N), jnp.bfloat16),
    ]
    for _ in range(C * 3):
        for rows in BUF_ROWS:
            scratch.append(pltpu.VMEM((rows, CN), jnp.bfloat16))
    for _ in range(C * 3):
        for rows in BUF_ROWS:
            scratch.append(pltpu.VMEM((rows, CN), jnp.bfloat16))
    scratch.append(pltpu.SemaphoreType.DMA((C * SEMS_PER_CHUNK,)))
    scratch.append(pltpu.SemaphoreType.DMA((C * SEMS_PER_CHUNK,)))

    return pl.pallas_call(
        body,
        out_shape=jax.ShapeDtypeStruct((M, N), jnp.bfloat16),
        in_specs=[
            pl.BlockSpec(memory_space=pltpu.VMEM),
            pl.BlockSpec(memory_space=pltpu.VMEM),
        ],
        out_specs=pl.BlockSpec(memory_space=pltpu.VMEM),
        scratch_shapes=scratch,
        compiler_params=pltpu.CompilerParams(collective_id=0),
    )(A, B)
